# baseline (speedup 1.0000x reference)
import functools

import jax
import jax.numpy as jnp
from jax import lax
from jax.experimental import pallas as pl
from jax.experimental.pallas import tpu as pltpu
from jax.experimental.pallas import tpu_sc as plsc

N_NODES = 10000
N_EDGES = 160000
D = 256
DQ = D // 2
NC = 2
NS = 16
EDGES_PER_TILE = N_EDGES // NS
CHUNK = 50
NCHUNK = EDGES_PER_TILE // CHUNK
STAGE = 40
NBUF = 4
G = 2
ROWS_PER_TILE = 624
TAIL_ROWS = N_NODES - NS * ROWS_PER_TILE
TAIL_OFF = NS * ROWS_PER_TILE


def _sc_body(x_all, src4, dst4, zeros, out0, out1,
             src_vm, dst_vm, rows0, rows1, rows2, rows3, agg_sh,
             gsem0, gsem1, gsem2, gsem3, ssem0, ssem1, ssem2, ssem3):
    c = lax.axis_index("c")
    s = lax.axis_index("s")
    r0 = s * ROWS_PER_TILE

    def zero_agg():
        pltpu.sync_copy(zeros, agg_sh.at[pl.ds(r0, ROWS_PER_TILE)])
        pl.when(s == 0)(lambda: pltpu.sync_copy(
            zeros.at[pl.ds(0, TAIL_ROWS)],
            agg_sh.at[pl.ds(TAIL_OFF, TAIL_ROWS)]))

    def accumulate(col0):
        table = x_all.at[:, pl.ds(col0, DQ)]
        bufs = (rows0, rows1, rows2, rows3)
        gsems = (gsem0, gsem1, gsem2, gsem3)
        ssems = (ssem0, ssem1, ssem2, ssem3)

        def gather(jj, b):
            return pltpu.async_copy(table.at[src_vm.at[jj]], bufs[b], gsems[b])

        def gather_wait(jj, b):
            pltpu.make_async_copy(table.at[src_vm.at[jj]], bufs[b],
                                  gsems[b]).wait()

        def scatter(jj, b):
            return pltpu.async_copy(bufs[b], agg_sh.at[dst_vm.at[jj]],
                                    ssems[b], add=True)

        def scatter_wait(jj, b):
            pltpu.make_async_copy(bufs[b], agg_sh.at[dst_vm.at[jj]],
                                  ssems[b]).wait()

        for h in range(NCHUNK // STAGE):
            pltpu.sync_copy(src4.at[s, pl.ds(h * STAGE, STAGE)], src_vm)
            pltpu.sync_copy(dst4.at[s, pl.ds(h * STAGE, STAGE)], dst_vm)
            for b in range(G):
                gather(b, b)
            for jj in range(NBUF):
                gather_wait(jj, jj)
                scatter(jj, jj)
                if jj >= G:
                    scatter_wait(jj - G, (jj + G) % NBUF)
                gather(jj + G, (jj + G) % NBUF)

            def outer(j, carry):
                for k in range(NBUF):
                    jj = NBUF * j + k
                    gather_wait(jj, k)
                    scatter(jj, k)
                    k2 = (k + G) % NBUF
                    @pl.when(jj + G < STAGE)
                    def _(jj=jj, k2=k2):
                        scatter_wait(jj - G, k2)
                        gather(jj + G, k2)
                return carry
            lax.fori_loop(1, STAGE // NBUF, outer, 0)
            for jj in range(STAGE - 2 * G, STAGE):
                scatter_wait(jj, jj % NBUF)

    def writeout(out_ref):
        pltpu.sync_copy(agg_sh.at[pl.ds(r0, ROWS_PER_TILE)],
                        out_ref.at[pl.ds(r0, ROWS_PER_TILE)])
        pl.when(s == 0)(lambda: pltpu.sync_copy(
            agg_sh.at[pl.ds(TAIL_OFF, TAIL_ROWS)],
            out_ref.at[pl.ds(TAIL_OFF, TAIL_ROWS)]))

    zero_agg()
    plsc.subcore_barrier()

    accumulate(c * DQ)
    plsc.subcore_barrier()
    pl.when(c == 0)(lambda: writeout(out0))
    pl.when(c == 1)(lambda: writeout(out1))


_quarter = jax.ShapeDtypeStruct((N_NODES, DQ), jnp.float32)
_sc_scatter = functools.partial(
    pl.kernel,
    out_type=(_quarter, _quarter),
    mesh=plsc.VectorSubcoreMesh(core_axis_name="c", subcore_axis_name="s",
                                num_cores=NC, num_subcores=NS),
    scratch_types=[
        pltpu.VMEM((STAGE, CHUNK), jnp.int32),
        pltpu.VMEM((STAGE, CHUNK), jnp.int32),
        pltpu.VMEM((CHUNK, DQ), jnp.float32),
        pltpu.VMEM((CHUNK, DQ), jnp.float32),
        pltpu.VMEM((CHUNK, DQ), jnp.float32),
        pltpu.VMEM((CHUNK, DQ), jnp.float32),
        pltpu.VMEM_SHARED((N_NODES, DQ), jnp.float32),
        pltpu.SemaphoreType.DMA,
        pltpu.SemaphoreType.DMA,
        pltpu.SemaphoreType.DMA,
        pltpu.SemaphoreType.DMA,
        pltpu.SemaphoreType.DMA,
        pltpu.SemaphoreType.DMA,
        pltpu.SemaphoreType.DMA,
        pltpu.SemaphoreType.DMA,
    ],
)(_sc_body)


def _mlp_body(x_ref, a0_ref, a1_ref,
              w1_ref, b1_ref, w2_ref, b2_ref, o_ref):
    h = x_ref[...] + jnp.concatenate([a0_ref[...], a1_ref[...]], axis=1)
    h = jnp.dot(h, w1_ref[...], preferred_element_type=jnp.float32) + b1_ref[...]
    h = jnp.maximum(h, 0.0)
    o_ref[...] = (jnp.dot(h, w2_ref[...], preferred_element_type=jnp.float32)
                  + b2_ref[...])


BLK = 1000


def _mlp(x, aggs, w1, b1, w2, b2):
    return pl.pallas_call(
        _mlp_body,
        grid=(N_NODES // BLK,),
        in_specs=[
            pl.BlockSpec((BLK, D), lambda i: (i, 0)),
            pl.BlockSpec((BLK, DQ), lambda i: (i, 0)),
            pl.BlockSpec((BLK, DQ), lambda i: (i, 0)),
            pl.BlockSpec((D, D), lambda i: (0, 0)),
            pl.BlockSpec((1, D), lambda i: (0, 0)),
            pl.BlockSpec((D, D), lambda i: (0, 0)),
            pl.BlockSpec((1, D), lambda i: (0, 0)),
        ],
        out_specs=pl.BlockSpec((BLK, D), lambda i: (i, 0)),
        out_shape=jax.ShapeDtypeStruct((N_NODES, D), jnp.float32),
    )(x, *aggs, w1, b1.reshape(1, D), w2, b2.reshape(1, D))


def kernel(x, edge_index, W1, b1, W2, b2):
    ei = edge_index.astype(jnp.int32).reshape(2, NS, NCHUNK, CHUNK)
    zeros = jnp.zeros((ROWS_PER_TILE, DQ), jnp.float32)
    aggs = _sc_scatter(x, ei[0], ei[1], zeros)
    return _mlp(x, aggs, W1, b1, W2, b2)


# speedup vs baseline: 1.0445x; 1.0445x over previous
import functools

import jax
import jax.numpy as jnp
from jax import lax
from jax.experimental import pallas as pl
from jax.experimental.pallas import tpu as pltpu
from jax.experimental.pallas import tpu_sc as plsc

N_NODES = 10000
N_EDGES = 160000
D = 256
DQ = D // 2
NC = 2
NS = 16
EDGES_PER_TILE = N_EDGES // NS
CHUNK = 125
NCHUNK = EDGES_PER_TILE // CHUNK
STAGE = 40
NBUF = 2
G = 1
ROWS_PER_TILE = 624
TAIL_ROWS = N_NODES - NS * ROWS_PER_TILE
TAIL_OFF = NS * ROWS_PER_TILE


def _sc_body(x_all, src4, dst4, zeros, out0, out1,
             src_vm, dst_vm, rows0, rows1, agg_sh,
             gsem0, gsem1, ssem0, ssem1):
    c = lax.axis_index("c")
    s = lax.axis_index("s")
    r0 = s * ROWS_PER_TILE

    def zero_agg():
        pltpu.sync_copy(zeros, agg_sh.at[pl.ds(r0, ROWS_PER_TILE)])
        pl.when(s == 0)(lambda: pltpu.sync_copy(
            zeros.at[pl.ds(0, TAIL_ROWS)],
            agg_sh.at[pl.ds(TAIL_OFF, TAIL_ROWS)]))

    def accumulate(col0):
        table = x_all.at[:, pl.ds(col0, DQ)]
        bufs = (rows0, rows1)
        gsems = (gsem0, gsem1)
        ssems = (ssem0, ssem1)

        def gather(jj, b):
            return pltpu.async_copy(table.at[src_vm.at[jj]], bufs[b], gsems[b])

        def gather_wait(jj, b):
            pltpu.make_async_copy(table.at[src_vm.at[jj]], bufs[b],
                                  gsems[b]).wait()

        def scatter(jj, b):
            return pltpu.async_copy(bufs[b], agg_sh.at[dst_vm.at[jj]],
                                    ssems[b], add=True)

        def scatter_wait(jj, b):
            pltpu.make_async_copy(bufs[b], agg_sh.at[dst_vm.at[jj]],
                                  ssems[b]).wait()

        for h in range(NCHUNK // STAGE):
            pltpu.sync_copy(src4.at[s, pl.ds(h * STAGE, STAGE)], src_vm)
            pltpu.sync_copy(dst4.at[s, pl.ds(h * STAGE, STAGE)], dst_vm)
            for b in range(G):
                gather(b, b)
            for jj in range(NBUF):
                gather_wait(jj, jj)
                scatter(jj, jj)
                if jj >= G:
                    scatter_wait(jj - G, (jj + G) % NBUF)
                gather(jj + G, (jj + G) % NBUF)

            def outer(j, carry):
                for k in range(NBUF):
                    jj = NBUF * j + k
                    gather_wait(jj, k)
                    scatter(jj, k)
                    k2 = (k + G) % NBUF
                    @pl.when(jj + G < STAGE)
                    def _(jj=jj, k2=k2):
                        scatter_wait(jj - G, k2)
                        gather(jj + G, k2)
                return carry
            lax.fori_loop(1, STAGE // NBUF, outer, 0)
            for jj in range(STAGE - 2 * G, STAGE):
                scatter_wait(jj, jj % NBUF)

    def writeout(out_ref):
        pltpu.sync_copy(agg_sh.at[pl.ds(r0, ROWS_PER_TILE)],
                        out_ref.at[pl.ds(r0, ROWS_PER_TILE)])
        pl.when(s == 0)(lambda: pltpu.sync_copy(
            agg_sh.at[pl.ds(TAIL_OFF, TAIL_ROWS)],
            out_ref.at[pl.ds(TAIL_OFF, TAIL_ROWS)]))

    zero_agg()
    plsc.subcore_barrier()

    accumulate(c * DQ)
    plsc.subcore_barrier()
    pl.when(c == 0)(lambda: writeout(out0))
    pl.when(c == 1)(lambda: writeout(out1))


_quarter = jax.ShapeDtypeStruct((N_NODES, DQ), jnp.float32)
_sc_scatter = functools.partial(
    pl.kernel,
    out_type=(_quarter, _quarter),
    mesh=plsc.VectorSubcoreMesh(core_axis_name="c", subcore_axis_name="s",
                                num_cores=NC, num_subcores=NS),
    scratch_types=[
        pltpu.VMEM((STAGE, CHUNK), jnp.int32),
        pltpu.VMEM((STAGE, CHUNK), jnp.int32),
        pltpu.VMEM((CHUNK, DQ), jnp.float32),
        pltpu.VMEM((CHUNK, DQ), jnp.float32),
        pltpu.VMEM_SHARED((N_NODES, DQ), jnp.float32),
        pltpu.SemaphoreType.DMA,
        pltpu.SemaphoreType.DMA,
        pltpu.SemaphoreType.DMA,
        pltpu.SemaphoreType.DMA,
    ],
)(_sc_body)


def _mlp_body(x_ref, a0_ref, a1_ref,
              w1_ref, b1_ref, w2_ref, b2_ref, o_ref):
    h = x_ref[...] + jnp.concatenate([a0_ref[...], a1_ref[...]], axis=1)
    h = jnp.dot(h, w1_ref[...], preferred_element_type=jnp.float32) + b1_ref[...]
    h = jnp.maximum(h, 0.0)
    o_ref[...] = (jnp.dot(h, w2_ref[...], preferred_element_type=jnp.float32)
                  + b2_ref[...])


BLK = 1000


def _mlp(x, aggs, w1, b1, w2, b2):
    return pl.pallas_call(
        _mlp_body,
        grid=(N_NODES // BLK,),
        in_specs=[
            pl.BlockSpec((BLK, D), lambda i: (i, 0)),
            pl.BlockSpec((BLK, DQ), lambda i: (i, 0)),
            pl.BlockSpec((BLK, DQ), lambda i: (i, 0)),
            pl.BlockSpec((D, D), lambda i: (0, 0)),
            pl.BlockSpec((1, D), lambda i: (0, 0)),
            pl.BlockSpec((D, D), lambda i: (0, 0)),
            pl.BlockSpec((1, D), lambda i: (0, 0)),
        ],
        out_specs=pl.BlockSpec((BLK, D), lambda i: (i, 0)),
        out_shape=jax.ShapeDtypeStruct((N_NODES, D), jnp.float32),
    )(x, *aggs, w1, b1.reshape(1, D), w2, b2.reshape(1, D))


def kernel(x, edge_index, W1, b1, W2, b2):
    ei = edge_index.astype(jnp.int32).reshape(2, NS, NCHUNK, CHUNK)
    zeros = jnp.zeros((ROWS_PER_TILE, DQ), jnp.float32)
    aggs = _sc_scatter(x, ei[0], ei[1], zeros)
    return _mlp(x, aggs, W1, b1, W2, b2)


# speedup vs baseline: 1.1805x; 1.1302x over previous
import functools

import jax
import jax.numpy as jnp
from jax import lax
from jax.experimental import pallas as pl
from jax.experimental.pallas import tpu as pltpu
from jax.experimental.pallas import tpu_sc as plsc

N_NODES = 10000
N_EDGES = 160000
D = 256
DQ = D // 2
NC = 2
NS = 16
EDGES_PER_TILE = N_EDGES // NS
CHUNK = 125
NCHUNK = EDGES_PER_TILE // CHUNK
HALF = NCHUNK // 2
ROWS_PER_TILE = 624
TAIL_ROWS = N_NODES - NS * ROWS_PER_TILE
TAIL_OFF = NS * ROWS_PER_TILE


def _sc_body(x_all, src4, dst4, zeros, out0, out1,
             src_vm, dst_vm, rows0, rows1, agg_sh, gsem0, gsem1):
    c = lax.axis_index("c")
    s = lax.axis_index("s")
    r0 = s * ROWS_PER_TILE

    def zero_agg():
        pltpu.sync_copy(zeros, agg_sh.at[pl.ds(r0, ROWS_PER_TILE)])
        pl.when(s == 0)(lambda: pltpu.sync_copy(
            zeros.at[pl.ds(0, TAIL_ROWS)],
            agg_sh.at[pl.ds(TAIL_OFF, TAIL_ROWS)]))

    def accumulate(col0):
        table = x_all.at[:, pl.ds(col0, DQ)]
        bufs = ((rows0, gsem0), (rows1, gsem1))
        for h in range(NCHUNK // HALF):
            pltpu.sync_copy(src4.at[s, pl.ds(h * HALF, HALF)], src_vm)
            pltpu.sync_copy(dst4.at[s, pl.ds(h * HALF, HALF)], dst_vm)
            for b, (buf, gsem) in enumerate(bufs):
                pltpu.async_copy(table.at[src_vm.at[b]], buf, gsem)

            def outer(j, carry):
                for b, (buf, gsem) in enumerate(bufs):
                    jj = 2 * j + b
                    pltpu.make_async_copy(
                        table.at[src_vm.at[jj]], buf, gsem).wait()
                    pltpu.sync_copy(buf, agg_sh.at[dst_vm.at[jj]], add=True)
                    @pl.when(jj + 2 < HALF)
                    def _(buf=buf, gsem=gsem, jj=jj):
                        pltpu.async_copy(table.at[src_vm.at[jj + 2]], buf, gsem)
                return carry
            lax.fori_loop(0, HALF // 2, outer, 0)

    def writeout(out_ref):
        pltpu.sync_copy(agg_sh.at[pl.ds(r0, ROWS_PER_TILE)],
                        out_ref.at[pl.ds(r0, ROWS_PER_TILE)])
        pl.when(s == 0)(lambda: pltpu.sync_copy(
            agg_sh.at[pl.ds(TAIL_OFF, TAIL_ROWS)],
            out_ref.at[pl.ds(TAIL_OFF, TAIL_ROWS)]))

    zero_agg()
    plsc.subcore_barrier()

    accumulate(c * DQ)
    plsc.subcore_barrier()
    pl.when(c == 0)(lambda: writeout(out0))
    pl.when(c == 1)(lambda: writeout(out1))


_quarter = jax.ShapeDtypeStruct((N_NODES, DQ), jnp.float32)
_sc_scatter = functools.partial(
    pl.kernel,
    out_type=(_quarter, _quarter),
    mesh=plsc.VectorSubcoreMesh(core_axis_name="c", subcore_axis_name="s",
                                num_cores=NC, num_subcores=NS),
    scratch_types=[
        pltpu.VMEM((HALF, CHUNK), jnp.int32),
        pltpu.VMEM((HALF, CHUNK), jnp.int32),
        pltpu.VMEM((CHUNK, DQ), jnp.float32),
        pltpu.VMEM((CHUNK, DQ), jnp.float32),
        pltpu.VMEM_SHARED((N_NODES, DQ), jnp.float32),
        pltpu.SemaphoreType.DMA,
        pltpu.SemaphoreType.DMA,
    ],
)(_sc_body)


def _mlp_body(x_ref, a0_ref, a1_ref,
              w1_ref, b1_ref, w2_ref, b2_ref, o_ref):
    h = x_ref[...] + jnp.concatenate([a0_ref[...], a1_ref[...]], axis=1)
    h = jnp.dot(h.astype(jnp.bfloat16), w1_ref[...],
                preferred_element_type=jnp.float32) + b1_ref[...]
    h = jnp.maximum(h, 0.0)
    o_ref[...] = jnp.dot(h.astype(jnp.bfloat16), w2_ref[...],
                         preferred_element_type=jnp.float32) + b2_ref[...]


BLK = 1000


def _mlp(x, aggs, w1, b1, w2, b2):
    return pl.pallas_call(
        _mlp_body,
        grid=(N_NODES // BLK,),
        in_specs=[
            pl.BlockSpec((BLK, D), lambda i: (i, 0)),
            pl.BlockSpec((BLK, DQ), lambda i: (i, 0)),
            pl.BlockSpec((BLK, DQ), lambda i: (i, 0)),
            pl.BlockSpec((D, D), lambda i: (0, 0)),
            pl.BlockSpec((1, D), lambda i: (0, 0)),
            pl.BlockSpec((D, D), lambda i: (0, 0)),
            pl.BlockSpec((1, D), lambda i: (0, 0)),
        ],
        out_specs=pl.BlockSpec((BLK, D), lambda i: (i, 0)),
        out_shape=jax.ShapeDtypeStruct((N_NODES, D), jnp.float32),
    )(x, *aggs, w1.astype(jnp.bfloat16), b1.reshape(1, D),
      w2.astype(jnp.bfloat16), b2.reshape(1, D))


def kernel(x, edge_index, W1, b1, W2, b2):
    ei = edge_index.astype(jnp.int32).reshape(2, NS, NCHUNK, CHUNK)
    zeros = jnp.zeros((ROWS_PER_TILE, DQ), jnp.float32)
    aggs = _sc_scatter(x, ei[0], ei[1], zeros)
    return _mlp(x, aggs, W1, b1, W2, b2)


# speedup vs baseline: 1.1882x; 1.0065x over previous
import functools

import jax
import jax.numpy as jnp
from jax import lax
from jax.experimental import pallas as pl
from jax.experimental.pallas import tpu as pltpu
from jax.experimental.pallas import tpu_sc as plsc

N_NODES = 10000
N_EDGES = 160000
D = 256
DQ = D // 2
NC = 2
NS = 16
EDGES_PER_TILE = N_EDGES // NS
CHUNK = 125
NCHUNK = EDGES_PER_TILE // CHUNK
HALF = NCHUNK // 2
ROWS_PER_TILE = 624
TAIL_ROWS = N_NODES - NS * ROWS_PER_TILE
TAIL_OFF = NS * ROWS_PER_TILE


def _sc_body(x_all, src4, dst4, zeros, out0, out1,
             src_vm, dst_vm, rows0, rows1, agg_sh, gsem0, gsem1):
    c = lax.axis_index("c")
    s = lax.axis_index("s")
    r0 = s * ROWS_PER_TILE

    def zero_agg():
        pltpu.sync_copy(zeros, agg_sh.at[pl.ds(r0, ROWS_PER_TILE)])
        pl.when(s == 0)(lambda: pltpu.sync_copy(
            zeros.at[pl.ds(0, TAIL_ROWS)],
            agg_sh.at[pl.ds(TAIL_OFF, TAIL_ROWS)]))

    def accumulate(col0):
        table = x_all.at[:, pl.ds(col0, DQ)]
        bufs = ((rows0, gsem0), (rows1, gsem1))
        for h in range(NCHUNK // HALF):
            pltpu.sync_copy(src4.at[s, pl.ds(h * HALF, HALF)], src_vm)
            pltpu.sync_copy(dst4.at[s, pl.ds(h * HALF, HALF)], dst_vm)
            for b, (buf, gsem) in enumerate(bufs):
                pltpu.async_copy(table.at[src_vm.at[b]], buf, gsem)

            def outer(j, carry):
                for b, (buf, gsem) in enumerate(bufs):
                    jj = 2 * j + b
                    pltpu.make_async_copy(
                        table.at[src_vm.at[jj]], buf, gsem).wait()
                    pltpu.sync_copy(buf, agg_sh.at[dst_vm.at[jj]], add=True)
                    @pl.when(jj + 2 < HALF)
                    def _(buf=buf, gsem=gsem, jj=jj):
                        pltpu.async_copy(table.at[src_vm.at[jj + 2]], buf, gsem)
                return carry
            lax.fori_loop(0, HALF // 2, outer, 0)

    def writeout(out_ref):
        pltpu.sync_copy(agg_sh.at[pl.ds(r0, ROWS_PER_TILE)],
                        out_ref.at[pl.ds(r0, ROWS_PER_TILE)])
        pl.when(s == 0)(lambda: pltpu.sync_copy(
            agg_sh.at[pl.ds(TAIL_OFF, TAIL_ROWS)],
            out_ref.at[pl.ds(TAIL_OFF, TAIL_ROWS)]))

    zero_agg()
    plsc.subcore_barrier()

    accumulate(c * DQ)
    plsc.subcore_barrier()
    pl.when(c == 0)(lambda: writeout(out0))
    pl.when(c == 1)(lambda: writeout(out1))


_quarter = jax.ShapeDtypeStruct((N_NODES, DQ), jnp.float32)
_sc_scatter = functools.partial(
    pl.kernel,
    out_type=(_quarter, _quarter),
    mesh=plsc.VectorSubcoreMesh(core_axis_name="c", subcore_axis_name="s",
                                num_cores=NC, num_subcores=NS),
    scratch_types=[
        pltpu.VMEM((HALF, CHUNK), jnp.int32),
        pltpu.VMEM((HALF, CHUNK), jnp.int32),
        pltpu.VMEM((CHUNK, DQ), jnp.float32),
        pltpu.VMEM((CHUNK, DQ), jnp.float32),
        pltpu.VMEM_SHARED((N_NODES, DQ), jnp.float32),
        pltpu.SemaphoreType.DMA,
        pltpu.SemaphoreType.DMA,
    ],
)(_sc_body)


def _mlp_body(x_ref, a0_ref, a1_ref,
              w1_ref, b1_ref, w2_ref, b2_ref, o_ref):
    h = x_ref[...] + jnp.concatenate([a0_ref[...], a1_ref[...]], axis=1)
    h = jnp.dot(h, w1_ref[...], preferred_element_type=jnp.float32) + b1_ref[...]
    h = jnp.maximum(h, 0.0)
    o_ref[...] = (jnp.dot(h, w2_ref[...], preferred_element_type=jnp.float32)
                  + b2_ref[...])


BLK = 1000


def _mlp(x, aggs, w1, b1, w2, b2):
    return pl.pallas_call(
        _mlp_body,
        grid=(N_NODES // BLK,),
        in_specs=[
            pl.BlockSpec((BLK, D), lambda i: (i, 0)),
            pl.BlockSpec((BLK, DQ), lambda i: (i, 0)),
            pl.BlockSpec((BLK, DQ), lambda i: (i, 0)),
            pl.BlockSpec((D, D), lambda i: (0, 0)),
            pl.BlockSpec((1, D), lambda i: (0, 0)),
            pl.BlockSpec((D, D), lambda i: (0, 0)),
            pl.BlockSpec((1, D), lambda i: (0, 0)),
        ],
        out_specs=pl.BlockSpec((BLK, D), lambda i: (i, 0)),
        out_shape=jax.ShapeDtypeStruct((N_NODES, D), jnp.float32),
    )(x, *aggs, w1, b1.reshape(1, D), w2, b2.reshape(1, D))


def kernel(x, edge_index, W1, b1, W2, b2):
    ei = edge_index.astype(jnp.int32).reshape(2, NS, NCHUNK, CHUNK)
    zeros = jnp.zeros((ROWS_PER_TILE, DQ), jnp.float32)
    aggs = _sc_scatter(x, ei[0], ei[1], zeros)
    return _mlp(x, aggs, W1, b1, W2, b2)
